# baseline (speedup 1.0000x reference)
import jax
import jax.numpy as jnp
from jax.experimental import pallas as pl
from jax.experimental.pallas import tpu as pltpu

N = 4096
D = 128
B = 256
K = N // B
CE = 1024
NC = N // CE
_BN_SCALE = 1.0 / (1.0 + 1e-5) ** 0.5


def _linear_bn_relu(x, W, b, g, be):
    pre = jax.lax.dot_general(
        x.astype(jnp.bfloat16), W[...].astype(jnp.bfloat16),
        (((1,), (1,)), ((), ())),
        preferred_element_type=jnp.float32)
    s = g[...] * _BN_SCALE
    return jnp.maximum(pre * s + (b[...] * s + be[...]), 0.0)


def _gcn_body(A_blk, rhs1, W0, b0, Wb0, bb0, W1, b1,
              g0, be0, gb0, beb0, g1, be1,
              out, A_sc, rs_sc, aug_sc, h2_sc, tT_sc):
    r = pl.program_id(0)
    sl = pl.ds(r * B, B)
    blk_bf = A_blk[...].astype(jnp.bfloat16)
    A_sc[sl, :] = blk_bf
    p = jnp.dot(blk_bf, rhs1[...], preferred_element_type=jnp.float32)
    rs = jnp.maximum(p[:, D:], 1e-8)
    rs_sc[sl, :] = rs
    h1 = _linear_bn_relu(p[:, :D] / rs, W0, b0, g0, be0)
    aug_sc[sl, :D] = h1.astype(jnp.bfloat16)
    aug_sc[sl, D:] = jnp.ones((B, D), jnp.bfloat16)

    @pl.when(r == K - 1)
    def _():
        def l2a(i, c):
            ksl = pl.ds(i * CE, CE)
            aug_t = jnp.transpose(aug_sc[ksl, :])
            contrib = jnp.dot(aug_t, A_sc[ksl, :],
                              preferred_element_type=jnp.float32)
            @pl.when(i == 0)
            def _():
                tT_sc[...] = contrib

            @pl.when(i > 0)
            def _():
                tT_sc[...] += contrib
            return c
        jax.lax.fori_loop(0, NC, l2a, 0)
        xT = tT_sc[:D, :] / jnp.maximum(tT_sc[D:, :], 1e-8)
        h2_sc[...] = _linear_bn_relu(
            jnp.transpose(xT), Wb0, bb0, gb0, beb0).astype(jnp.bfloat16)

        def l3(i, c):
            csl = pl.ds(i * CE, CE)
            p3 = jnp.dot(A_sc[csl, :], h2_sc[...],
                         preferred_element_type=jnp.float32)
            out[csl, :] = _linear_bn_relu(p3 / rs_sc[csl, :], W1, b1, g1, be1)
            return c
        jax.lax.fori_loop(0, NC, l3, 0)


def kernel(H_source, H_target, A, W0, b0, Wb0, bb0, W1, b1,
           g0, be0, gb0, beb0, g1, be1):
    del H_target
    row = lambda v: v.reshape(1, D)
    vec_spec = pl.BlockSpec((1, D), lambda r: (0, 0))
    mat_spec = pl.BlockSpec((D, D), lambda r: (0, 0))
    call = pl.pallas_call(
        _gcn_body,
        grid=(K,),
        in_specs=[
            pl.BlockSpec((B, N), lambda r: (r, 0)),
            pl.BlockSpec((N, 2 * D), lambda r: (0, 0)),
            mat_spec, vec_spec,
            mat_spec, vec_spec,
            mat_spec, vec_spec,
            vec_spec, vec_spec,
            vec_spec, vec_spec,
            vec_spec, vec_spec,
        ],
        out_specs=pl.BlockSpec((N, D), lambda r: (0, 0)),
        out_shape=jax.ShapeDtypeStruct((N, D), jnp.float32),
        scratch_shapes=[
            pltpu.VMEM((N, N), jnp.bfloat16),
            pltpu.VMEM((N, D), jnp.float32),
            pltpu.VMEM((N, 2 * D), jnp.bfloat16),
            pltpu.VMEM((N, D), jnp.bfloat16),
            pltpu.VMEM((2 * D, N), jnp.float32),
        ],
        compiler_params=pltpu.CompilerParams(
            dimension_semantics=("arbitrary",),
        ),
    )
    rhs1 = jnp.concatenate(
        [H_source.astype(jnp.bfloat16),
         jnp.ones((N, D), jnp.bfloat16)], axis=1)
    return call(A, rhs1, W0, row(b0), Wb0, row(bb0), W1, row(b1),
                row(g0), row(be0), row(gb0), row(beb0), row(g1), row(be1))


# speedup vs baseline: 1.1070x; 1.1070x over previous
import jax
import jax.numpy as jnp
from jax.experimental import pallas as pl
from jax.experimental.pallas import tpu as pltpu

N = 4096
D = 128
B = 512
K = N // B
CE = 1024
NC = N // CE
_BN_SCALE = 1.0 / (1.0 + 1e-5) ** 0.5


def _linear_bn_relu(x, W, b, g, be):
    pre = jax.lax.dot_general(
        x.astype(jnp.bfloat16), W[...].astype(jnp.bfloat16),
        (((1,), (1,)), ((), ())),
        preferred_element_type=jnp.float32)
    s = g[...] * _BN_SCALE
    return jnp.maximum(pre * s + (b[...] * s + be[...]), 0.0)


def _gcn_body(A_blk, rhs1, W0, b0, Wb0, bb0, W1, b1,
              g0, be0, gb0, beb0, g1, be1,
              out, A_sc, rs_sc, aug_sc, h2_sc):
    r = pl.program_id(0)
    sl = pl.ds(r * B, B)
    blk_bf = A_blk[...].astype(jnp.bfloat16)
    A_sc[sl, :] = blk_bf
    p = jnp.dot(blk_bf, rhs1[...], preferred_element_type=jnp.float32)
    rs = jnp.maximum(p[:, D:], 1e-8)
    rs_sc[sl, :] = rs
    h1 = _linear_bn_relu(p[:, :D] / rs, W0, b0, g0, be0)
    aug_sc[sl, :D] = h1.astype(jnp.bfloat16)
    aug_sc[sl, D:] = jnp.ones((B, D), jnp.bfloat16)

    @pl.when(r == K - 1)
    def _():
        def l2(i, c):
            csl = pl.ds(i * CE, CE)
            t = jax.lax.dot_general(
                A_sc[:, csl], aug_sc[...], (((0,), (0,)), ((), ())),
                preferred_element_type=jnp.float32)
            x = t[:, :D] / jnp.maximum(t[:, D:], 1e-8)
            h2 = _linear_bn_relu(x, Wb0, bb0, gb0, beb0)
            h2_sc[csl, :] = h2.astype(jnp.bfloat16)
            return c
        jax.lax.fori_loop(0, NC, l2, 0)

        def l3(i, c):
            csl = pl.ds(i * CE, CE)
            p3 = jnp.dot(A_sc[csl, :], h2_sc[...],
                         preferred_element_type=jnp.float32)
            out[csl, :] = _linear_bn_relu(p3 / rs_sc[csl, :], W1, b1, g1, be1)
            return c
        jax.lax.fori_loop(0, NC, l3, 0)


def kernel(H_source, H_target, A, W0, b0, Wb0, bb0, W1, b1,
           g0, be0, gb0, beb0, g1, be1):
    del H_target
    row = lambda v: v.reshape(1, D)
    vec_spec = pl.BlockSpec((1, D), lambda r: (0, 0))
    mat_spec = pl.BlockSpec((D, D), lambda r: (0, 0))
    call = pl.pallas_call(
        _gcn_body,
        grid=(K,),
        in_specs=[
            pl.BlockSpec((B, N), lambda r: (r, 0)),
            pl.BlockSpec((N, 2 * D), lambda r: (0, 0)),
            mat_spec, vec_spec,
            mat_spec, vec_spec,
            mat_spec, vec_spec,
            vec_spec, vec_spec,
            vec_spec, vec_spec,
            vec_spec, vec_spec,
        ],
        out_specs=pl.BlockSpec((N, D), lambda r: (0, 0)),
        out_shape=jax.ShapeDtypeStruct((N, D), jnp.float32),
        scratch_shapes=[
            pltpu.VMEM((N, N), jnp.bfloat16),
            pltpu.VMEM((N, D), jnp.float32),
            pltpu.VMEM((N, 2 * D), jnp.bfloat16),
            pltpu.VMEM((N, D), jnp.bfloat16),
        ],
        compiler_params=pltpu.CompilerParams(
            dimension_semantics=("arbitrary",),
        ),
    )
    rhs1 = jnp.concatenate(
        [H_source.astype(jnp.bfloat16),
         jnp.ones((N, D), jnp.bfloat16)], axis=1)
    return call(A, rhs1, W0, row(b0), Wb0, row(bb0), W1, row(b1),
                row(g0), row(be0), row(gb0), row(beb0), row(g1), row(be1))
